# baseline (speedup 1.0000x reference)
import dataclasses
import functools

import jax
import jax.numpy as jnp
from jax import lax
from jax.experimental import pallas as pl
from jax.experimental.pallas import tpu as pltpu
from jax.experimental.pallas import tpu_sc as plsc

N = 10000
E = 320000
IN_DIM = 128
HID = 16
HEADS = 8
NUM_LAYERS = 4
EDGE_DIM = 16
HD = HID * HEADS

NP = 10008
TRASH = N
K = 128
NCORE = 2
NSUB = 16
NTILE = NCORE * NSUB
ETOT = E + N
CHUNKS = 2624
CPT = CHUNKS // NTILE
SELF_CHUNK0 = E // K
NB = NP // 8

_HIGH = lax.Precision.HIGHEST


def _dot(a, b):
    return jnp.dot(a, b, preferred_element_type=jnp.float32, precision=_HIGH)


def _k_in_body(x_ref, w_ref, b_ref, g_ref, bb_ref, o_ref):
    h = _dot(x_ref[...], w_ref[...]) + b_ref[...]
    m = jnp.mean(h, axis=-1, keepdims=True)
    v = jnp.mean((h - m) ** 2, axis=-1, keepdims=True)
    h = (h - m) / jnp.sqrt(v + 1e-5) * g_ref[...] + bb_ref[...]
    o_ref[...] = jnp.maximum(h, 0.0)


def _k_pre_body(h_ref, w_ref, as_ref, ad_ref, hw_ref, s16_ref, d16_ref):
    hw = _dot(h_ref[...], w_ref[...])
    hw_ref[...] = hw
    s16_ref[...] = _dot(hw, as_ref[...])
    d16_ref[...] = _dot(hw, ad_ref[...])


def _k_edge_body(ea_ref, we_ref, p_ref):
    p_ref[...] = _dot(ea_ref[...], we_ref[...])


def _k_mean_body(ea_ref, o_ref):
    @pl.when(pl.program_id(0) == 0)
    def _():
        o_ref[...] = jnp.zeros_like(o_ref)

    s = jnp.sum(ea_ref[...], axis=0, keepdims=True)
    o_ref[...] += jnp.broadcast_to(s, o_ref.shape)


def _k_pad_body(ms_ref, we_ref, o_ref):
    m = ms_ref[0:1, :] * (1.0 / E)
    for l in range(NUM_LAYERS):
        v = _dot(m, we_ref[l])
        o_ref[l] = jnp.broadcast_to(v, (K, HEADS))


def _k_post_body(acc_ref, den_ref, r_ref, b_ref, g_ref, bb_ref, res_ref,
                 o_ref, *, use_res):
    a = acc_ref[0] + acc_ref[1]
    den = den_ref[0] + den_ref[1] + 1e-16
    dene = _dot(den, r_ref[...])
    o = a / dene + b_ref[...]
    m = jnp.mean(o, axis=-1, keepdims=True)
    v = jnp.mean((o - m) ** 2, axis=-1, keepdims=True)
    o = (o - m) / jnp.sqrt(v + 1e-5) * g_ref[...] + bb_ref[...]
    if use_res:
        o = o + res_ref[...]
    o_ref[...] = jnp.where(o > 0, o, jnp.exp(o) - 1.0)


def _k_final_body(acc_ref, den_ref, r_ref, m_ref, b_ref, g_ref, bb_ref, o_ref):
    a = acc_ref[0] + acc_ref[1]
    den = den_ref[0] + den_ref[1] + 1e-16
    dene = _dot(den, r_ref[...])
    o = _dot(a / dene, m_ref[...]) + b_ref[...]
    m = jnp.mean(o, axis=-1, keepdims=True)
    v = jnp.mean((o - m) ** 2, axis=-1, keepdims=True)
    o_ref[...] = (o - m) / jnp.sqrt(v + 1e-5) * g_ref[...] + bb_ref[...]


_ROWS_B = 1000
_GRID_N = N // _ROWS_B
_EB = 4000
_GRID_E = E // _EB


def _full(shape):
    return pl.BlockSpec(shape, lambda i: (0,) * len(shape))


def _rows(shape):
    return pl.BlockSpec(shape, lambda i: (i,) + (0,) * (len(shape) - 1))


def _sc_layer(hw, a_src16, a_dst16, edge_index, p_l, aepad_l):
    mesh = plsc.VectorSubcoreMesh(
        core_axis_name="c", subcore_axis_name="s",
        num_cores=NCORE, num_subcores=NSUB)

    cp = pltpu.CompilerParams()
    if "needs_layout_passes" in pltpu.CompilerParams.__dataclass_fields__:
        cp = dataclasses.replace(cp, needs_layout_passes=False)
    if "use_tc_tiling_on_sc" in pltpu.CompilerParams.__dataclass_fields__:
        cp = dataclasses.replace(cp, use_tc_tiling_on_sc=False)

    @functools.partial(
        pl.kernel,
        out_type=[
            jax.ShapeDtypeStruct((NCORE, NP, HD), jnp.float32),
            jax.ShapeDtypeStruct((NCORE, NP, EDGE_DIM), jnp.float32),
        ],
        mesh=mesh,
        scratch_types=[
            pltpu.VMEM_SHARED((NP, HD), jnp.float32),
            pltpu.VMEM_SHARED((NP, EDGE_DIM), jnp.float32),
            [pltpu.VMEM((K, HD), jnp.float32)] * 2,
            pltpu.VMEM((K, EDGE_DIM), jnp.float32),
            [pltpu.VMEM((K, EDGE_DIM), jnp.float32)] * 2,
            pltpu.VMEM((K * HEADS + 16,), jnp.float32),
            [pltpu.VMEM((K,), jnp.int32)] * 2,
            [pltpu.VMEM((K,), jnp.int32)] * 2,
            [pltpu.SemaphoreType.DMA] * 2,
            [pltpu.SemaphoreType.DMA] * 2,
            [pltpu.SemaphoreType.DMA] * 2,
        ],
        compiler_params=cp,
    )
    def k(hw_hbm, as_hbm, ad_hbm, ei_hbm, p_hbm, aep_hbm, acc_out, den_out,
          acc_sp, den_sp, hbs, ab, bbs, aeb, sbs, dbs,
          sem_i, sem_g, sem_s):
        c = lax.axis_index("c")
        s = lax.axis_index("s")
        w = c * NSUB + s

        @pl.loop(0, 8)
        def _(i):
            for j in range(HD // 16):
                hbs[0][i, pl.ds(j * 16, 16)] = jnp.zeros((16,), jnp.float32)
            bbs[0][i, :] = jnp.zeros((16,), jnp.float32)

        @pl.loop(s, NB, step=NSUB)
        def _(blk):
            pltpu.sync_copy(hbs[0].at[pl.ds(0, 8)],
                            acc_sp.at[pl.ds(blk * 8, 8)])
            pltpu.sync_copy(bbs[0].at[pl.ds(0, 8)],
                            den_sp.at[pl.ds(blk * 8, 8)])

        plsc.subcore_barrier()

        def idx_copies(t, b):
            base = (w * CPT + t) * K
            return [
                (ei_hbm.at[0, pl.ds(base, K)], sbs[b], sem_i[b]),
                (ei_hbm.at[1, pl.ds(base, K)], dbs[b], sem_i[b]),
            ]

        def start_idx(t, b):
            tc = w * CPT + t
            base = tc * K

            @pl.when(tc < SELF_CHUNK0)
            def _():
                for src, dst, sem in idx_copies(t, b):
                    pltpu.async_copy(src, dst, sem)

            @pl.when(tc >= SELF_CHUNK0)
            def _():
                @pl.loop(0, K // 16)
                def _(j):
                    v = (base - E + j * 16) + lax.iota(jnp.int32, 16)
                    sbs[b][pl.ds(j * 16, 16)] = jnp.minimum(v, N - 1)
                    dbs[b][pl.ds(j * 16, 16)] = jnp.minimum(v, TRASH)

        def wait_idx(t, b):
            tc = w * CPT + t

            @pl.when(tc < SELF_CHUNK0)
            def _():
                for src, dst, sem in idx_copies(t, b):
                    pltpu.make_async_copy(src, dst, sem).wait()

        def gat_copies(b):
            return [
                (as_hbm.at[sbs[b]], ab, sem_g[b]),
                (ad_hbm.at[dbs[b]], bbs[b], sem_g[b]),
                (hw_hbm.at[sbs[b]], hbs[b], sem_g[b]),
            ]

        def start_gat(t, b):
            tc = w * CPT + t
            for src, dst, sem in gat_copies(b):
                pltpu.async_copy(src, dst, sem)

            @pl.when(tc < SELF_CHUNK0)
            def _():
                pltpu.async_copy(p_hbm.at[pl.ds(tc * K * HEADS, K * HEADS)],
                                 aeb.at[pl.ds(0, K * HEADS)], sem_g[b])

            @pl.when(tc >= SELF_CHUNK0)
            def _():
                pltpu.async_copy(aep_hbm, aeb.at[pl.ds(0, K * HEADS)],
                                 sem_g[b])

        def wait_gat(b):
            for src, dst, sem in gat_copies(b):
                pltpu.make_async_copy(src, dst, sem).wait()
            pltpu.make_async_copy(aep_hbm, aeb.at[pl.ds(0, K * HEADS)],
                                  sem_g[b]).wait()

        def sc_copies(b):
            return [
                (hbs[b], acc_sp.at[dbs[b]], sem_s[b]),
                (bbs[b], den_sp.at[dbs[b]], sem_s[b]),
            ]

        lane = lax.iota(jnp.int32, 16)

        def compute(b):
            @pl.loop(0, K)
            def _(i):
                al = ab[i, :] + bbs[b][i, :] + aeb[pl.ds(i * HEADS, 16)]
                al = jnp.where(al >= 0.0, al, al * 0.2)
                ex = jnp.exp(al)
                bbs[b][i, :] = ex
                for j in range(HEADS):
                    sj = jnp.sum(jnp.where(lane == j, ex, 0.0))
                    sl = pl.ds(j * 16, 16)
                    hbs[b][i, sl] = hbs[b][i, sl] * sj

        start_idx(0, 0)
        wait_idx(0, 0)
        start_gat(0, 0)

        @pl.loop(0, CPT, step=2)
        def _(t0):
            for b in range(2):
                t = t0 + b
                b2 = 1 - b

                @pl.when(t >= 1)
                def _():
                    for src, dst, sem in sc_copies(b2):
                        pltpu.make_async_copy(src, dst, sem).wait()

                @pl.when(t + 1 < CPT)
                def _():
                    start_idx(t + 1, b2)

                wait_gat(b)

                compute(b)

                for src, dst, sem in sc_copies(b):
                    pltpu.async_copy(src, dst, sem, add=True)

                @pl.when(t + 1 < CPT)
                def _():
                    wait_idx(t + 1, b2)
                    start_gat(t + 1, b2)

        for src, dst, sem in sc_copies(1):
            pltpu.make_async_copy(src, dst, sem).wait()

        plsc.subcore_barrier()

        @pl.loop(s, NB, step=NSUB)
        def _(blk):
            pltpu.sync_copy(acc_sp.at[pl.ds(blk * 8, 8)],
                            acc_out.at[c, pl.ds(blk * 8, 8)])
            pltpu.sync_copy(den_sp.at[pl.ds(blk * 8, 8)],
                            den_out.at[c, pl.ds(blk * 8, 8)])

    return k(hw, a_src16, a_dst16, edge_index, p_l, aepad_l)


def _att_fold(att):
    flat = att[0].reshape(HD)
    h_of = jnp.arange(HD, dtype=jnp.int32) // HID
    return flat[:, None] * jax.nn.one_hot(h_of, EDGE_DIM, dtype=jnp.float32)


def kernel(x, edge_index, edge_attr, params):
    layers = params["layers"]

    we_all = jnp.stack([
        jnp.sum(p["W_edge"].reshape(EDGE_DIM, HEADS, HID)
                * p["att_edge"][0][None], axis=-1)
        for p in layers])
    a_src_m = [_att_fold(p["att_src"]) for p in layers]
    a_dst_m = [_att_fold(p["att_dst"]) for p in layers]

    h_of = jnp.arange(HD, dtype=jnp.int32) // HID
    d_of = jnp.arange(HD, dtype=jnp.int32) % HID
    rmat = jax.nn.one_hot(h_of, EDGE_DIM, dtype=jnp.float32).T
    mmat = jax.nn.one_hot(d_of, HID, dtype=jnp.float32) / HEADS

    r2 = lambda v: v.reshape(1, -1)

    h = pl.pallas_call(
        _k_in_body,
        grid=(_GRID_N,),
        in_specs=[_rows((_ROWS_B, IN_DIM)), _full((IN_DIM, HD)),
                  _full((1, HD)), _full((1, HD)), _full((1, HD))],
        out_specs=_rows((_ROWS_B, HD)),
        out_shape=jax.ShapeDtypeStruct((N, HD), jnp.float32),
    )(x, params["W_in"], r2(params["b_in"]),
      r2(params["ln_in_g"]), r2(params["ln_in_b"]))

    p_all = [
        pl.pallas_call(
            _k_edge_body,
            grid=(_GRID_E,),
            in_specs=[_rows((_EB, EDGE_DIM)), _full((EDGE_DIM, HEADS))],
            out_specs=_rows((_EB, HEADS)),
            out_shape=jax.ShapeDtypeStruct((E, HEADS), jnp.float32),
        )(edge_attr, we_all[l])
        for l in range(NUM_LAYERS)
    ]

    msum = pl.pallas_call(
        _k_mean_body,
        grid=(_GRID_E,),
        in_specs=[_rows((_EB, EDGE_DIM))],
        out_specs=_full((8, EDGE_DIM)),
        out_shape=jax.ShapeDtypeStruct((8, EDGE_DIM), jnp.float32),
    )(edge_attr)

    aepad = pl.pallas_call(
        _k_pad_body,
        in_specs=[pl.BlockSpec((8, EDGE_DIM), lambda: (0, 0)),
                  pl.BlockSpec((NUM_LAYERS, EDGE_DIM, HEADS),
                               lambda: (0, 0, 0))],
        out_specs=pl.BlockSpec((NUM_LAYERS, K, HEADS), lambda: (0, 0, 0)),
        out_shape=jax.ShapeDtypeStruct((NUM_LAYERS, K, HEADS), jnp.float32),
    )(msum, we_all)
    aepad = aepad.reshape(NUM_LAYERS, K * HEADS)

    for l in range(NUM_LAYERS):
        p = layers[l]
        hw, s16, d16 = pl.pallas_call(
            _k_pre_body,
            grid=(_GRID_N,),
            in_specs=[_rows((_ROWS_B, HD)), _full((HD, HD)),
                      _full((HD, EDGE_DIM)), _full((HD, EDGE_DIM))],
            out_specs=[_rows((_ROWS_B, HD)), _rows((_ROWS_B, EDGE_DIM)),
                       _rows((_ROWS_B, EDGE_DIM))],
            out_shape=[jax.ShapeDtypeStruct((N, HD), jnp.float32),
                       jax.ShapeDtypeStruct((N, EDGE_DIM), jnp.float32),
                       jax.ShapeDtypeStruct((N, EDGE_DIM), jnp.float32)],
        )(h, p["W"], a_src_m[l], a_dst_m[l])

        d16p = jnp.concatenate(
            [d16, jnp.zeros((NP - N, EDGE_DIM), jnp.float32)], axis=0)

        acc, den = _sc_layer(hw, s16, d16p, edge_index,
                             p_all[l].reshape(-1), aepad[l])

        if l < NUM_LAYERS - 1:
            h = pl.pallas_call(
                functools.partial(_k_post_body, use_res=(l > 0)),
                grid=(_GRID_N,),
                in_specs=[
                    pl.BlockSpec((NCORE, _ROWS_B, HD), lambda i: (0, i, 0)),
                    pl.BlockSpec((NCORE, _ROWS_B, EDGE_DIM),
                                 lambda i: (0, i, 0)),
                    _full((EDGE_DIM, HD)), _full((1, HD)), _full((1, HD)),
                    _full((1, HD)), _rows((_ROWS_B, HD))],
                out_specs=_rows((_ROWS_B, HD)),
                out_shape=jax.ShapeDtypeStruct((N, HD), jnp.float32),
            )(acc, den, rmat, r2(p["bias"]), r2(p["ln_g"]), r2(p["ln_b"]), h)
        else:
            h = pl.pallas_call(
                _k_final_body,
                grid=(_GRID_N,),
                in_specs=[
                    pl.BlockSpec((NCORE, _ROWS_B, HD), lambda i: (0, i, 0)),
                    pl.BlockSpec((NCORE, _ROWS_B, EDGE_DIM),
                                 lambda i: (0, i, 0)),
                    _full((EDGE_DIM, HD)), _full((HD, HID)),
                    _full((1, HID)), _full((1, HID)), _full((1, HID))],
                out_specs=_rows((_ROWS_B, HID)),
                out_shape=jax.ShapeDtypeStruct((N, HID), jnp.float32),
            )(acc, den, rmat, mmat, r2(p["bias"]), r2(p["ln_g"]),
              r2(p["ln_b"]))

    return h


# speedup vs baseline: 1.3148x; 1.3148x over previous
import dataclasses
import functools

import jax
import jax.numpy as jnp
from jax import lax
from jax.experimental import pallas as pl
from jax.experimental.pallas import tpu as pltpu
from jax.experimental.pallas import tpu_sc as plsc

N = 10000
E = 320000
IN_DIM = 128
HID = 16
HEADS = 8
NUM_LAYERS = 4
EDGE_DIM = 16
HD = HID * HEADS

NP = 10008
TRASH = N
K = 128
NCORE = 2
NSUB = 16
NTILE = NCORE * NSUB
ETOT = E + N
CHUNKS = 2624
CPT = CHUNKS // NTILE
SELF_CHUNK0 = E // K
NB = NP // 8

_HIGH = lax.Precision.HIGHEST


def _dot(a, b):
    return jnp.dot(a, b, preferred_element_type=jnp.float32, precision=_HIGH)


def _k_in_body(x_ref, w_ref, b_ref, g_ref, bb_ref, o_ref):
    h = _dot(x_ref[...], w_ref[...]) + b_ref[...]
    m = jnp.mean(h, axis=-1, keepdims=True)
    v = jnp.mean((h - m) ** 2, axis=-1, keepdims=True)
    h = (h - m) / jnp.sqrt(v + 1e-5) * g_ref[...] + bb_ref[...]
    o_ref[...] = jnp.maximum(h, 0.0)


def _k_pre_body(h_ref, w_ref, as_ref, ad_ref, hw_ref, s16_ref, d16_ref):
    hw = _dot(h_ref[...], w_ref[...])
    hw_ref[...] = hw
    s16_ref[...] = _dot(hw, as_ref[...])
    d16_ref[...] = _dot(hw, ad_ref[...])


def _k_edge_body(ea_ref, we_ref, p_ref):
    p_ref[...] = _dot(ea_ref[...], we_ref[...])


def _k_mean_body(ea_ref, o_ref):
    @pl.when(pl.program_id(0) == 0)
    def _():
        o_ref[...] = jnp.zeros_like(o_ref)

    s = jnp.sum(ea_ref[...], axis=0, keepdims=True)
    o_ref[...] += jnp.broadcast_to(s, o_ref.shape)


def _k_pad_body(ms_ref, we_ref, o_ref):
    m = ms_ref[0:1, :] * (1.0 / E)
    for l in range(NUM_LAYERS):
        v = _dot(m, we_ref[l])
        o_ref[l] = jnp.broadcast_to(v, (K, HEADS))


def _k_post_body(acc_ref, den_ref, r_ref, b_ref, g_ref, bb_ref, res_ref,
                 o_ref, *, use_res):
    a = acc_ref[0] + acc_ref[1]
    den = den_ref[0] + den_ref[1] + 1e-16
    dene = _dot(den, r_ref[...])
    o = a / dene + b_ref[...]
    m = jnp.mean(o, axis=-1, keepdims=True)
    v = jnp.mean((o - m) ** 2, axis=-1, keepdims=True)
    o = (o - m) / jnp.sqrt(v + 1e-5) * g_ref[...] + bb_ref[...]
    if use_res:
        o = o + res_ref[...]
    o_ref[...] = jnp.where(o > 0, o, jnp.exp(o) - 1.0)


def _k_final_body(acc_ref, den_ref, r_ref, m_ref, b_ref, g_ref, bb_ref, o_ref):
    a = acc_ref[0] + acc_ref[1]
    den = den_ref[0] + den_ref[1] + 1e-16
    dene = _dot(den, r_ref[...])
    o = _dot(a / dene, m_ref[...]) + b_ref[...]
    m = jnp.mean(o, axis=-1, keepdims=True)
    v = jnp.mean((o - m) ** 2, axis=-1, keepdims=True)
    o_ref[...] = (o - m) / jnp.sqrt(v + 1e-5) * g_ref[...] + bb_ref[...]


_ROWS_B = 1000
_GRID_N = N // _ROWS_B
_EB = 4000
_GRID_E = E // _EB


def _full(shape):
    return pl.BlockSpec(shape, lambda i: (0,) * len(shape))


def _rows(shape):
    return pl.BlockSpec(shape, lambda i: (i,) + (0,) * (len(shape) - 1))


def _sc_layer(hw, a_src16, a_dst16, edge_index, p_l, aepad_l):
    mesh = plsc.VectorSubcoreMesh(
        core_axis_name="c", subcore_axis_name="s",
        num_cores=NCORE, num_subcores=NSUB)

    cp = pltpu.CompilerParams()
    if "needs_layout_passes" in pltpu.CompilerParams.__dataclass_fields__:
        cp = dataclasses.replace(cp, needs_layout_passes=False)
    if "use_tc_tiling_on_sc" in pltpu.CompilerParams.__dataclass_fields__:
        cp = dataclasses.replace(cp, use_tc_tiling_on_sc=False)

    @functools.partial(
        pl.kernel,
        out_type=[
            jax.ShapeDtypeStruct((NCORE, NP, HD), jnp.float32),
            jax.ShapeDtypeStruct((NCORE, NP, EDGE_DIM), jnp.float32),
        ],
        mesh=mesh,
        scratch_types=[
            pltpu.VMEM_SHARED((NP, HD), jnp.float32),
            pltpu.VMEM_SHARED((NP, EDGE_DIM), jnp.float32),
            [pltpu.VMEM((K, HD), jnp.float32)] * 2,
            pltpu.VMEM((K, EDGE_DIM), jnp.float32),
            [pltpu.VMEM((K, EDGE_DIM), jnp.float32)] * 2,
            pltpu.VMEM((K * HEADS + 16,), jnp.float32),
            [pltpu.VMEM((K,), jnp.int32)] * 2,
            [pltpu.VMEM((K,), jnp.int32)] * 2,
            [pltpu.SemaphoreType.DMA] * 2,
            [pltpu.SemaphoreType.DMA] * 2,
            [pltpu.SemaphoreType.DMA] * 2,
        ],
        compiler_params=cp,
    )
    def k(hw_hbm, as_hbm, ad_hbm, ei_hbm, p_hbm, aep_hbm, acc_out, den_out,
          acc_sp, den_sp, hbs, ab, bbs, aeb, sbs, dbs,
          sem_i, sem_g, sem_s):
        c = lax.axis_index("c")
        s = lax.axis_index("s")
        w = c * NSUB + s

        @pl.loop(0, 8)
        def _(i):
            for j in range(HD // 16):
                hbs[0][i, pl.ds(j * 16, 16)] = jnp.zeros((16,), jnp.float32)
            bbs[0][i, :] = jnp.zeros((16,), jnp.float32)

        @pl.loop(s, NB, step=NSUB)
        def _(blk):
            pltpu.sync_copy(hbs[0].at[pl.ds(0, 8)],
                            acc_sp.at[pl.ds(blk * 8, 8)])
            pltpu.sync_copy(bbs[0].at[pl.ds(0, 8)],
                            den_sp.at[pl.ds(blk * 8, 8)])

        plsc.subcore_barrier()

        def idx_copies(t, b):
            base = (w * CPT + t) * K
            return [
                (ei_hbm.at[0, pl.ds(base, K)], sbs[b], sem_i[b]),
                (ei_hbm.at[1, pl.ds(base, K)], dbs[b], sem_i[b]),
            ]

        def start_idx(t, b):
            tc = w * CPT + t
            base = tc * K

            @pl.when(tc < SELF_CHUNK0)
            def _():
                for src, dst, sem in idx_copies(t, b):
                    pltpu.async_copy(src, dst, sem)

            @pl.when(tc >= SELF_CHUNK0)
            def _():
                @pl.loop(0, K // 16)
                def _(j):
                    v = (base - E + j * 16) + lax.iota(jnp.int32, 16)
                    sbs[b][pl.ds(j * 16, 16)] = jnp.minimum(v, N - 1)
                    dbs[b][pl.ds(j * 16, 16)] = jnp.minimum(v, TRASH)

        def wait_idx(t, b):
            tc = w * CPT + t

            @pl.when(tc < SELF_CHUNK0)
            def _():
                for src, dst, sem in idx_copies(t, b):
                    pltpu.make_async_copy(src, dst, sem).wait()

        def gat_copies(b):
            return [
                (as_hbm.at[sbs[b]], ab, sem_g[b]),
                (ad_hbm.at[dbs[b]], bbs[b], sem_g[b]),
                (hw_hbm.at[sbs[b]], hbs[b], sem_g[b]),
            ]

        def start_gat(t, b):
            tc = w * CPT + t
            for src, dst, sem in gat_copies(b):
                pltpu.async_copy(src, dst, sem)

            @pl.when(tc < SELF_CHUNK0)
            def _():
                pltpu.async_copy(p_hbm.at[pl.ds(tc * K * HEADS, K * HEADS)],
                                 aeb.at[pl.ds(0, K * HEADS)], sem_g[b])

            @pl.when(tc >= SELF_CHUNK0)
            def _():
                pltpu.async_copy(aep_hbm, aeb.at[pl.ds(0, K * HEADS)],
                                 sem_g[b])

        def wait_gat(b):
            for src, dst, sem in gat_copies(b):
                pltpu.make_async_copy(src, dst, sem).wait()
            pltpu.make_async_copy(aep_hbm, aeb.at[pl.ds(0, K * HEADS)],
                                  sem_g[b]).wait()

        def sc_copies(b):
            return [
                (hbs[b], acc_sp.at[dbs[b]], sem_s[b]),
                (bbs[b], den_sp.at[dbs[b]], sem_s[b]),
            ]

        _dn = lax.GatherDimensionNumbers(
            offset_dims=(), collapsed_slice_dims=(0,), start_index_map=(0,))

        def _splat(v, j):
            idx = jnp.full((16, 1), j, dtype=jnp.int32)
            return lax.gather(v, idx, _dn, (1,),
                              mode=lax.GatherScatterMode.PROMISE_IN_BOUNDS)

        def compute(b):
            @pl.loop(0, K)
            def _(i):
                al = ab[i, :] + bbs[b][i, :] + aeb[pl.ds(i * HEADS, 16)]
                al = jnp.where(al >= 0.0, al, al * 0.2)
                ex = jnp.exp(al)
                bbs[b][i, :] = ex
                for j in range(HEADS):
                    sl = pl.ds(j * 16, 16)
                    hbs[b][i, sl] = hbs[b][i, sl] * _splat(ex, j)

        start_idx(0, 0)
        wait_idx(0, 0)
        start_gat(0, 0)

        @pl.loop(0, CPT, step=2)
        def _(t0):
            for b in range(2):
                t = t0 + b
                b2 = 1 - b

                @pl.when(t >= 1)
                def _():
                    for src, dst, sem in sc_copies(b2):
                        pltpu.make_async_copy(src, dst, sem).wait()

                @pl.when(t + 1 < CPT)
                def _():
                    start_idx(t + 1, b2)

                wait_gat(b)

                compute(b)

                for src, dst, sem in sc_copies(b):
                    pltpu.async_copy(src, dst, sem, add=True)

                @pl.when(t + 1 < CPT)
                def _():
                    wait_idx(t + 1, b2)
                    start_gat(t + 1, b2)

        for src, dst, sem in sc_copies(1):
            pltpu.make_async_copy(src, dst, sem).wait()

        plsc.subcore_barrier()

        @pl.loop(s, NB, step=NSUB)
        def _(blk):
            pltpu.sync_copy(acc_sp.at[pl.ds(blk * 8, 8)],
                            acc_out.at[c, pl.ds(blk * 8, 8)])
            pltpu.sync_copy(den_sp.at[pl.ds(blk * 8, 8)],
                            den_out.at[c, pl.ds(blk * 8, 8)])

    return k(hw, a_src16, a_dst16, edge_index, p_l, aepad_l)


def _att_fold(att):
    flat = att[0].reshape(HD)
    h_of = jnp.arange(HD, dtype=jnp.int32) // HID
    return flat[:, None] * jax.nn.one_hot(h_of, EDGE_DIM, dtype=jnp.float32)


def kernel(x, edge_index, edge_attr, params):
    layers = params["layers"]

    we_all = jnp.stack([
        jnp.sum(p["W_edge"].reshape(EDGE_DIM, HEADS, HID)
                * p["att_edge"][0][None], axis=-1)
        for p in layers])
    a_src_m = [_att_fold(p["att_src"]) for p in layers]
    a_dst_m = [_att_fold(p["att_dst"]) for p in layers]

    h_of = jnp.arange(HD, dtype=jnp.int32) // HID
    d_of = jnp.arange(HD, dtype=jnp.int32) % HID
    rmat = jax.nn.one_hot(h_of, EDGE_DIM, dtype=jnp.float32).T
    mmat = jax.nn.one_hot(d_of, HID, dtype=jnp.float32) / HEADS

    r2 = lambda v: v.reshape(1, -1)

    h = pl.pallas_call(
        _k_in_body,
        grid=(_GRID_N,),
        in_specs=[_rows((_ROWS_B, IN_DIM)), _full((IN_DIM, HD)),
                  _full((1, HD)), _full((1, HD)), _full((1, HD))],
        out_specs=_rows((_ROWS_B, HD)),
        out_shape=jax.ShapeDtypeStruct((N, HD), jnp.float32),
    )(x, params["W_in"], r2(params["b_in"]),
      r2(params["ln_in_g"]), r2(params["ln_in_b"]))

    p_all = [
        pl.pallas_call(
            _k_edge_body,
            grid=(_GRID_E,),
            in_specs=[_rows((_EB, EDGE_DIM)), _full((EDGE_DIM, HEADS))],
            out_specs=_rows((_EB, HEADS)),
            out_shape=jax.ShapeDtypeStruct((E, HEADS), jnp.float32),
        )(edge_attr, we_all[l])
        for l in range(NUM_LAYERS)
    ]

    msum = pl.pallas_call(
        _k_mean_body,
        grid=(_GRID_E,),
        in_specs=[_rows((_EB, EDGE_DIM))],
        out_specs=_full((8, EDGE_DIM)),
        out_shape=jax.ShapeDtypeStruct((8, EDGE_DIM), jnp.float32),
    )(edge_attr)

    aepad = pl.pallas_call(
        _k_pad_body,
        in_specs=[pl.BlockSpec((8, EDGE_DIM), lambda: (0, 0)),
                  pl.BlockSpec((NUM_LAYERS, EDGE_DIM, HEADS),
                               lambda: (0, 0, 0))],
        out_specs=pl.BlockSpec((NUM_LAYERS, K, HEADS), lambda: (0, 0, 0)),
        out_shape=jax.ShapeDtypeStruct((NUM_LAYERS, K, HEADS), jnp.float32),
    )(msum, we_all)
    aepad = aepad.reshape(NUM_LAYERS, K * HEADS)

    for l in range(NUM_LAYERS):
        p = layers[l]
        hw, s16, d16 = pl.pallas_call(
            _k_pre_body,
            grid=(_GRID_N,),
            in_specs=[_rows((_ROWS_B, HD)), _full((HD, HD)),
                      _full((HD, EDGE_DIM)), _full((HD, EDGE_DIM))],
            out_specs=[_rows((_ROWS_B, HD)), _rows((_ROWS_B, EDGE_DIM)),
                       _rows((_ROWS_B, EDGE_DIM))],
            out_shape=[jax.ShapeDtypeStruct((N, HD), jnp.float32),
                       jax.ShapeDtypeStruct((N, EDGE_DIM), jnp.float32),
                       jax.ShapeDtypeStruct((N, EDGE_DIM), jnp.float32)],
        )(h, p["W"], a_src_m[l], a_dst_m[l])

        d16p = jnp.concatenate(
            [d16, jnp.zeros((NP - N, EDGE_DIM), jnp.float32)], axis=0)

        acc, den = _sc_layer(hw, s16, d16p, edge_index,
                             p_all[l].reshape(-1), aepad[l])

        if l < NUM_LAYERS - 1:
            h = pl.pallas_call(
                functools.partial(_k_post_body, use_res=(l > 0)),
                grid=(_GRID_N,),
                in_specs=[
                    pl.BlockSpec((NCORE, _ROWS_B, HD), lambda i: (0, i, 0)),
                    pl.BlockSpec((NCORE, _ROWS_B, EDGE_DIM),
                                 lambda i: (0, i, 0)),
                    _full((EDGE_DIM, HD)), _full((1, HD)), _full((1, HD)),
                    _full((1, HD)), _rows((_ROWS_B, HD))],
                out_specs=_rows((_ROWS_B, HD)),
                out_shape=jax.ShapeDtypeStruct((N, HD), jnp.float32),
            )(acc, den, rmat, r2(p["bias"]), r2(p["ln_g"]), r2(p["ln_b"]), h)
        else:
            h = pl.pallas_call(
                _k_final_body,
                grid=(_GRID_N,),
                in_specs=[
                    pl.BlockSpec((NCORE, _ROWS_B, HD), lambda i: (0, i, 0)),
                    pl.BlockSpec((NCORE, _ROWS_B, EDGE_DIM),
                                 lambda i: (0, i, 0)),
                    _full((EDGE_DIM, HD)), _full((HD, HID)),
                    _full((1, HID)), _full((1, HID)), _full((1, HID))],
                out_specs=_rows((_ROWS_B, HID)),
                out_shape=jax.ShapeDtypeStruct((N, HID), jnp.float32),
            )(acc, den, rmat, mmat, r2(p["bias"]), r2(p["ln_g"]),
              r2(p["ln_b"]))

    return h


# speedup vs baseline: 1.4655x; 1.1147x over previous
import dataclasses
import functools

import jax
import jax.numpy as jnp
from jax import lax
from jax.experimental import pallas as pl
from jax.experimental.pallas import tpu as pltpu
from jax.experimental.pallas import tpu_sc as plsc

N = 10000
E = 320000
IN_DIM = 128
HID = 16
HEADS = 8
NUM_LAYERS = 4
EDGE_DIM = 16
HD = HID * HEADS

NP = 10008
TRASH = N
K = 128
NCORE = 2
NSUB = 16
NTILE = NCORE * NSUB
ETOT = E + N
CHUNKS = 2624
CPT = CHUNKS // NTILE
SELF_CHUNK0 = E // K
NB = NP // 8

_HIGH = lax.Precision.HIGHEST


def _dot(a, b):
    return jnp.dot(a, b, preferred_element_type=jnp.float32, precision=_HIGH)


def _k_in_body(x_ref, w_ref, b_ref, g_ref, bb_ref, o_ref):
    h = _dot(x_ref[...], w_ref[...]) + b_ref[...]
    m = jnp.mean(h, axis=-1, keepdims=True)
    v = jnp.mean((h - m) ** 2, axis=-1, keepdims=True)
    h = (h - m) / jnp.sqrt(v + 1e-5) * g_ref[...] + bb_ref[...]
    o_ref[...] = jnp.maximum(h, 0.0)


def _k_pre_body(h_ref, w_ref, as_ref, ad_ref, hw_ref, s16_ref, d16_ref):
    hw = _dot(h_ref[...], w_ref[...])
    hw_ref[...] = hw
    s16_ref[...] = _dot(hw, as_ref[...])
    d16_ref[...] = _dot(hw, ad_ref[...])


def _k_edge_body(ea_ref, we_ref, p_ref):
    p_ref[...] = _dot(ea_ref[...], we_ref[...])


def _k_mean_body(ea_ref, o_ref):
    @pl.when(pl.program_id(0) == 0)
    def _():
        o_ref[...] = jnp.zeros_like(o_ref)

    s = jnp.sum(ea_ref[...], axis=0, keepdims=True)
    o_ref[...] += jnp.broadcast_to(s, o_ref.shape)


def _k_pad_body(ms_ref, we_ref, o_ref):
    m = ms_ref[0:1, :] * (1.0 / E)
    for l in range(NUM_LAYERS):
        v = _dot(m, we_ref[l])
        o_ref[l] = jnp.broadcast_to(v, (K, HEADS))


def _k_post_body(acc_ref, den_ref, r_ref, b_ref, g_ref, bb_ref, res_ref,
                 o_ref, *, use_res):
    a = acc_ref[0] + acc_ref[1]
    den = den_ref[0] + den_ref[1] + 1e-16
    dene = _dot(den, r_ref[...])
    o = a / dene + b_ref[...]
    m = jnp.mean(o, axis=-1, keepdims=True)
    v = jnp.mean((o - m) ** 2, axis=-1, keepdims=True)
    o = (o - m) / jnp.sqrt(v + 1e-5) * g_ref[...] + bb_ref[...]
    if use_res:
        o = o + res_ref[...]
    o_ref[...] = jnp.where(o > 0, o, jnp.exp(o) - 1.0)


def _k_final_body(acc_ref, den_ref, r_ref, m_ref, b_ref, g_ref, bb_ref, o_ref):
    a = acc_ref[0] + acc_ref[1]
    den = den_ref[0] + den_ref[1] + 1e-16
    dene = _dot(den, r_ref[...])
    o = _dot(a / dene, m_ref[...]) + b_ref[...]
    m = jnp.mean(o, axis=-1, keepdims=True)
    v = jnp.mean((o - m) ** 2, axis=-1, keepdims=True)
    o_ref[...] = (o - m) / jnp.sqrt(v + 1e-5) * g_ref[...] + bb_ref[...]


_ROWS_B = 1000
_GRID_N = N // _ROWS_B
_EB = 4000
_GRID_E = E // _EB


def _full(shape):
    return pl.BlockSpec(shape, lambda i: (0,) * len(shape))


def _rows(shape):
    return pl.BlockSpec(shape, lambda i: (i,) + (0,) * (len(shape) - 1))


def _sc_layer(hw, a_src16, a_dst16, edge_index, p_l, aepad_l):
    mesh = plsc.VectorSubcoreMesh(
        core_axis_name="c", subcore_axis_name="s",
        num_cores=NCORE, num_subcores=NSUB)

    cp = pltpu.CompilerParams()
    if "needs_layout_passes" in pltpu.CompilerParams.__dataclass_fields__:
        cp = dataclasses.replace(cp, needs_layout_passes=False)
    if "use_tc_tiling_on_sc" in pltpu.CompilerParams.__dataclass_fields__:
        cp = dataclasses.replace(cp, use_tc_tiling_on_sc=False)

    @functools.partial(
        pl.kernel,
        out_type=[
            jax.ShapeDtypeStruct((NCORE, NP, HD), jnp.float32),
            jax.ShapeDtypeStruct((NCORE, NP, EDGE_DIM), jnp.float32),
        ],
        mesh=mesh,
        scratch_types=[
            pltpu.VMEM_SHARED((NP, HD), jnp.float32),
            pltpu.VMEM_SHARED((NP, EDGE_DIM), jnp.float32),
            [pltpu.VMEM((K, HD), jnp.float32)] * 2,
            pltpu.VMEM((K, EDGE_DIM), jnp.float32),
            [pltpu.VMEM((K, EDGE_DIM), jnp.float32)] * 2,
            pltpu.VMEM((K * HEADS + 16,), jnp.float32),
            [pltpu.VMEM((K,), jnp.int32)] * 2,
            [pltpu.VMEM((K,), jnp.int32)] * 2,
            [pltpu.SemaphoreType.DMA] * 2,
            [pltpu.SemaphoreType.DMA] * 2,
            [pltpu.SemaphoreType.DMA] * 2,
        ],
        compiler_params=cp,
    )
    def k(hw_hbm, as_hbm, ad_hbm, ei_hbm, p_hbm, aep_hbm, acc_out, den_out,
          acc_sp, den_sp, hbs, ab, bbs, aeb, sbs, dbs,
          sem_i, sem_g, sem_s):
        c = lax.axis_index("c")
        s = lax.axis_index("s")
        w = c * NSUB + s

        @pl.loop(0, 8)
        def _(i):
            for j in range(HD // 16):
                hbs[0][i, pl.ds(j * 16, 16)] = jnp.zeros((16,), jnp.float32)
            bbs[0][i, :] = jnp.zeros((16,), jnp.float32)

        @pl.loop(s, NB, step=NSUB)
        def _(blk):
            pltpu.sync_copy(hbs[0].at[pl.ds(0, 8)],
                            acc_sp.at[pl.ds(blk * 8, 8)])
            pltpu.sync_copy(bbs[0].at[pl.ds(0, 8)],
                            den_sp.at[pl.ds(blk * 8, 8)])

        plsc.subcore_barrier()

        def idx_copies(t, b):
            base = (w + t * NTILE) * K
            return [
                (ei_hbm.at[0, pl.ds(base, K)], sbs[b], sem_i[b]),
                (ei_hbm.at[1, pl.ds(base, K)], dbs[b], sem_i[b]),
            ]

        def start_idx(t, b):
            tc = w + t * NTILE
            base = tc * K

            @pl.when(tc < SELF_CHUNK0)
            def _():
                for src, dst, sem in idx_copies(t, b):
                    pltpu.async_copy(src, dst, sem)

            @pl.when(tc >= SELF_CHUNK0)
            def _():
                @pl.loop(0, K // 16)
                def _(j):
                    v = (base - E + j * 16) + lax.iota(jnp.int32, 16)
                    sbs[b][pl.ds(j * 16, 16)] = jnp.minimum(v, N - 1)
                    dbs[b][pl.ds(j * 16, 16)] = jnp.minimum(v, TRASH)

        def wait_idx(t, b):
            tc = w + t * NTILE

            @pl.when(tc < SELF_CHUNK0)
            def _():
                for src, dst, sem in idx_copies(t, b):
                    pltpu.make_async_copy(src, dst, sem).wait()

        def gat_copies(b):
            return [
                (as_hbm.at[sbs[b]], ab, sem_g[b]),
                (ad_hbm.at[dbs[b]], bbs[b], sem_g[b]),
                (hw_hbm.at[sbs[b]], hbs[b], sem_g[b]),
            ]

        def start_gat(t, b):
            tc = w + t * NTILE
            for src, dst, sem in gat_copies(b):
                pltpu.async_copy(src, dst, sem)

            @pl.when(tc < SELF_CHUNK0)
            def _():
                pltpu.async_copy(p_hbm.at[pl.ds(tc * K * HEADS, K * HEADS)],
                                 aeb.at[pl.ds(0, K * HEADS)], sem_g[b])

            @pl.when(tc >= SELF_CHUNK0)
            def _():
                pltpu.async_copy(aep_hbm, aeb.at[pl.ds(0, K * HEADS)],
                                 sem_g[b])

        def wait_gat(b):
            for src, dst, sem in gat_copies(b):
                pltpu.make_async_copy(src, dst, sem).wait()
            pltpu.make_async_copy(aep_hbm, aeb.at[pl.ds(0, K * HEADS)],
                                  sem_g[b]).wait()

        def sc_copies(b):
            return [
                (hbs[b], acc_sp.at[dbs[b]], sem_s[b]),
                (bbs[b], den_sp.at[dbs[b]], sem_s[b]),
            ]

        _dn = lax.GatherDimensionNumbers(
            offset_dims=(), collapsed_slice_dims=(0,), start_index_map=(0,))

        def _splat(v, j):
            idx = jnp.full((16, 1), j, dtype=jnp.int32)
            return lax.gather(v, idx, _dn, (1,),
                              mode=lax.GatherScatterMode.PROMISE_IN_BOUNDS)

        def compute(b):
            @pl.loop(0, K)
            def _(i):
                al = ab[i, :] + bbs[b][i, :] + aeb[pl.ds(i * HEADS, 16)]
                al = jnp.where(al >= 0.0, al, al * 0.2)
                ex = jnp.exp(al)
                bbs[b][i, :] = ex
                for j in range(HEADS):
                    sl = pl.ds(j * 16, 16)
                    hbs[b][i, sl] = hbs[b][i, sl] * _splat(ex, j)

        start_idx(0, 0)
        wait_idx(0, 0)
        start_gat(0, 0)

        @pl.loop(0, CPT, step=2)
        def _(t0):
            for b in range(2):
                t = t0 + b
                b2 = 1 - b

                @pl.when(t >= 1)
                def _():
                    for src, dst, sem in sc_copies(b2):
                        pltpu.make_async_copy(src, dst, sem).wait()

                @pl.when(t + 1 < CPT)
                def _():
                    start_idx(t + 1, b2)

                wait_gat(b)

                compute(b)

                for src, dst, sem in sc_copies(b):
                    pltpu.async_copy(src, dst, sem, add=True)

                @pl.when(t + 1 < CPT)
                def _():
                    wait_idx(t + 1, b2)
                    start_gat(t + 1, b2)

        for src, dst, sem in sc_copies(1):
            pltpu.make_async_copy(src, dst, sem).wait()

        plsc.subcore_barrier()

        @pl.loop(s, NB, step=NSUB)
        def _(blk):
            pltpu.sync_copy(acc_sp.at[pl.ds(blk * 8, 8)],
                            acc_out.at[c, pl.ds(blk * 8, 8)])
            pltpu.sync_copy(den_sp.at[pl.ds(blk * 8, 8)],
                            den_out.at[c, pl.ds(blk * 8, 8)])

    return k(hw, a_src16, a_dst16, edge_index, p_l, aepad_l)


def _att_fold(att):
    flat = att[0].reshape(HD)
    h_of = jnp.arange(HD, dtype=jnp.int32) // HID
    return flat[:, None] * jax.nn.one_hot(h_of, EDGE_DIM, dtype=jnp.float32)


def kernel(x, edge_index, edge_attr, params):
    layers = params["layers"]

    we_all = jnp.stack([
        jnp.sum(p["W_edge"].reshape(EDGE_DIM, HEADS, HID)
                * p["att_edge"][0][None], axis=-1)
        for p in layers])
    a_src_m = [_att_fold(p["att_src"]) for p in layers]
    a_dst_m = [_att_fold(p["att_dst"]) for p in layers]

    h_of = jnp.arange(HD, dtype=jnp.int32) // HID
    d_of = jnp.arange(HD, dtype=jnp.int32) % HID
    rmat = jax.nn.one_hot(h_of, EDGE_DIM, dtype=jnp.float32).T
    mmat = jax.nn.one_hot(d_of, HID, dtype=jnp.float32) / HEADS

    r2 = lambda v: v.reshape(1, -1)

    h = pl.pallas_call(
        _k_in_body,
        grid=(_GRID_N,),
        in_specs=[_rows((_ROWS_B, IN_DIM)), _full((IN_DIM, HD)),
                  _full((1, HD)), _full((1, HD)), _full((1, HD))],
        out_specs=_rows((_ROWS_B, HD)),
        out_shape=jax.ShapeDtypeStruct((N, HD), jnp.float32),
    )(x, params["W_in"], r2(params["b_in"]),
      r2(params["ln_in_g"]), r2(params["ln_in_b"]))

    p_all = [
        pl.pallas_call(
            _k_edge_body,
            grid=(_GRID_E,),
            in_specs=[_rows((_EB, EDGE_DIM)), _full((EDGE_DIM, HEADS))],
            out_specs=_rows((_EB, HEADS)),
            out_shape=jax.ShapeDtypeStruct((E, HEADS), jnp.float32),
        )(edge_attr, we_all[l])
        for l in range(NUM_LAYERS)
    ]

    msum = pl.pallas_call(
        _k_mean_body,
        grid=(_GRID_E,),
        in_specs=[_rows((_EB, EDGE_DIM))],
        out_specs=_full((8, EDGE_DIM)),
        out_shape=jax.ShapeDtypeStruct((8, EDGE_DIM), jnp.float32),
    )(edge_attr)

    aepad = pl.pallas_call(
        _k_pad_body,
        in_specs=[pl.BlockSpec((8, EDGE_DIM), lambda: (0, 0)),
                  pl.BlockSpec((NUM_LAYERS, EDGE_DIM, HEADS),
                               lambda: (0, 0, 0))],
        out_specs=pl.BlockSpec((NUM_LAYERS, K, HEADS), lambda: (0, 0, 0)),
        out_shape=jax.ShapeDtypeStruct((NUM_LAYERS, K, HEADS), jnp.float32),
    )(msum, we_all)
    aepad = aepad.reshape(NUM_LAYERS, K * HEADS)

    for l in range(NUM_LAYERS):
        p = layers[l]
        hw, s16, d16 = pl.pallas_call(
            _k_pre_body,
            grid=(_GRID_N,),
            in_specs=[_rows((_ROWS_B, HD)), _full((HD, HD)),
                      _full((HD, EDGE_DIM)), _full((HD, EDGE_DIM))],
            out_specs=[_rows((_ROWS_B, HD)), _rows((_ROWS_B, EDGE_DIM)),
                       _rows((_ROWS_B, EDGE_DIM))],
            out_shape=[jax.ShapeDtypeStruct((N, HD), jnp.float32),
                       jax.ShapeDtypeStruct((N, EDGE_DIM), jnp.float32),
                       jax.ShapeDtypeStruct((N, EDGE_DIM), jnp.float32)],
        )(h, p["W"], a_src_m[l], a_dst_m[l])

        d16p = jnp.concatenate(
            [d16, jnp.zeros((NP - N, EDGE_DIM), jnp.float32)], axis=0)

        acc, den = _sc_layer(hw, s16, d16p, edge_index,
                             p_all[l].reshape(-1), aepad[l])

        if l < NUM_LAYERS - 1:
            h = pl.pallas_call(
                functools.partial(_k_post_body, use_res=(l > 0)),
                grid=(_GRID_N,),
                in_specs=[
                    pl.BlockSpec((NCORE, _ROWS_B, HD), lambda i: (0, i, 0)),
                    pl.BlockSpec((NCORE, _ROWS_B, EDGE_DIM),
                                 lambda i: (0, i, 0)),
                    _full((EDGE_DIM, HD)), _full((1, HD)), _full((1, HD)),
                    _full((1, HD)), _rows((_ROWS_B, HD))],
                out_specs=_rows((_ROWS_B, HD)),
                out_shape=jax.ShapeDtypeStruct((N, HD), jnp.float32),
            )(acc, den, rmat, r2(p["bias"]), r2(p["ln_g"]), r2(p["ln_b"]), h)
        else:
            h = pl.pallas_call(
                _k_final_body,
                grid=(_GRID_N,),
                in_specs=[
                    pl.BlockSpec((NCORE, _ROWS_B, HD), lambda i: (0, i, 0)),
                    pl.BlockSpec((NCORE, _ROWS_B, EDGE_DIM),
                                 lambda i: (0, i, 0)),
                    _full((EDGE_DIM, HD)), _full((HD, HID)),
                    _full((1, HID)), _full((1, HID)), _full((1, HID))],
                out_specs=_rows((_ROWS_B, HID)),
                out_shape=jax.ShapeDtypeStruct((N, HID), jnp.float32),
            )(acc, den, rmat, mmat, r2(p["bias"]), r2(p["ln_g"]),
              r2(p["ln_b"]))

    return h


# speedup vs baseline: 1.5900x; 1.0850x over previous
import dataclasses
import functools

import jax
import jax.numpy as jnp
from jax import lax
from jax.experimental import pallas as pl
from jax.experimental.pallas import tpu as pltpu
from jax.experimental.pallas import tpu_sc as plsc

N = 10000
E = 320000
IN_DIM = 128
HID = 16
HEADS = 8
NUM_LAYERS = 4
EDGE_DIM = 16
HD = HID * HEADS

NP = 10008
TRASH = N
K = 128
NCORE = 2
NSUB = 16
NTILE = NCORE * NSUB
ETOT = E + N
CHUNKS = 2624
CPT = CHUNKS // NTILE
SELF_CHUNK0 = E // K
ROWS_A = 632
ROWS_LAST = NP - 15 * ROWS_A

_HIGH = lax.Precision.HIGHEST


def _dot(a, b):
    return jnp.dot(a, b, preferred_element_type=jnp.float32, precision=_HIGH)


def _layer_norm(h, g, bb):
    m = jnp.mean(h, axis=-1, keepdims=True)
    v = jnp.mean((h - m) ** 2, axis=-1, keepdims=True)
    return (h - m) / jnp.sqrt(v + 1e-5) * g + bb


def _k_in_body(x_ref, w_ref, b_ref, g_ref, bb_ref, wn_ref, as_ref, ad_ref,
               h_ref, hw_ref, s16_ref, d16_ref):
    h = jnp.maximum(
        _layer_norm(_dot(x_ref[...], w_ref[...]) + b_ref[...],
                    g_ref[...], bb_ref[...]), 0.0)
    h_ref[...] = h
    hw = _dot(h, wn_ref[...])
    hw_ref[...] = hw
    s16_ref[...] = _dot(hw, as_ref[...])
    d16_ref[...] = _dot(hw, ad_ref[...])


def _k_edge_body(ea_ref, we_ref, p_ref):
    p_ref[...] = _dot(ea_ref[...], we_ref[...])


def _k_mean_body(ea_ref, o_ref):
    @pl.when(pl.program_id(0) == 0)
    def _():
        o_ref[...] = jnp.zeros_like(o_ref)

    s = jnp.sum(ea_ref[...], axis=0, keepdims=True)
    o_ref[...] += jnp.broadcast_to(s, o_ref.shape)


def _k_pad_body(ms_ref, we_ref, o_ref):
    m = ms_ref[0:1, :] * (1.0 / E)
    for l in range(NUM_LAYERS):
        v = _dot(m, we_ref[l])
        o_ref[l] = jnp.broadcast_to(v, (K, HEADS))


def _k_fused_body(acc_ref, den_ref, r_ref, b_ref, g_ref, bb_ref, res_ref,
                  wn_ref, as_ref, ad_ref,
                  h_ref, hw_ref, s16_ref, d16_ref, *, use_res):
    a = acc_ref[0] + acc_ref[1]
    den = den_ref[0] + den_ref[1] + 1e-16
    dene = _dot(den, r_ref[...])
    o = _layer_norm(a / dene + b_ref[...], g_ref[...], bb_ref[...])
    if use_res:
        o = o + res_ref[...]
    h = jnp.where(o > 0, o, jnp.exp(o) - 1.0)
    h_ref[...] = h
    hw = _dot(h, wn_ref[...])
    hw_ref[...] = hw
    s16_ref[...] = _dot(hw, as_ref[...])
    d16_ref[...] = _dot(hw, ad_ref[...])


def _k_final_body(acc_ref, den_ref, r_ref, m_ref, b_ref, g_ref, bb_ref,
                  o_ref):
    a = acc_ref[0] + acc_ref[1]
    den = den_ref[0] + den_ref[1] + 1e-16
    dene = _dot(den, r_ref[...])
    o = _dot(a / dene, m_ref[...]) + b_ref[...]
    o_ref[...] = _layer_norm(o, g_ref[...], bb_ref[...])


_ROWS_B = 1000
_GRID_N = N // _ROWS_B
_EB = 4000
_GRID_E = E // _EB


def _full(shape):
    return pl.BlockSpec(shape, lambda i: (0,) * len(shape))


def _rows(shape):
    return pl.BlockSpec(shape, lambda i: (i,) + (0,) * (len(shape) - 1))


def _sc_layer(hw, a_src16, a_dst16, edge_index, p_l, aepad_l, z128, z16):
    mesh = plsc.VectorSubcoreMesh(
        core_axis_name="c", subcore_axis_name="s",
        num_cores=NCORE, num_subcores=NSUB)

    cp = pltpu.CompilerParams()
    if "needs_layout_passes" in pltpu.CompilerParams.__dataclass_fields__:
        cp = dataclasses.replace(cp, needs_layout_passes=False)
    if "use_tc_tiling_on_sc" in pltpu.CompilerParams.__dataclass_fields__:
        cp = dataclasses.replace(cp, use_tc_tiling_on_sc=False)

    @functools.partial(
        pl.kernel,
        out_type=[
            jax.ShapeDtypeStruct((NCORE, NP, HD), jnp.float32),
            jax.ShapeDtypeStruct((NCORE, NP, EDGE_DIM), jnp.float32),
        ],
        mesh=mesh,
        scratch_types=[
            pltpu.VMEM_SHARED((NP, HD), jnp.float32),
            pltpu.VMEM_SHARED((NP, EDGE_DIM), jnp.float32),
            [pltpu.VMEM((K, HD), jnp.float32)] * 2,
            pltpu.VMEM((K, EDGE_DIM), jnp.float32),
            [pltpu.VMEM((K, EDGE_DIM), jnp.float32)] * 2,
            pltpu.VMEM((K * HEADS + 16,), jnp.float32),
            [pltpu.VMEM((K,), jnp.int32)] * 2,
            [pltpu.VMEM((K,), jnp.int32)] * 2,
            [pltpu.SemaphoreType.DMA] * 2,
            [pltpu.SemaphoreType.DMA] * 2,
            [pltpu.SemaphoreType.DMA] * 2,
        ],
        compiler_params=cp,
    )
    def k(hw_hbm, as_hbm, ad_hbm, ei_hbm, p_hbm, aep_hbm, z128_hbm, z16_hbm,
          acc_out, den_out,
          acc_sp, den_sp, hbs, ab, bbs, aeb, sbs, dbs,
          sem_i, sem_g, sem_s):
        c = lax.axis_index("c")
        s = lax.axis_index("s")
        w = c * NSUB + s

        @pl.when(s < NSUB - 1)
        def _():
            r0 = s * ROWS_A
            pltpu.sync_copy(z128_hbm.at[pl.ds(r0, ROWS_A)],
                            acc_sp.at[pl.ds(r0, ROWS_A)])
            pltpu.sync_copy(z16_hbm.at[pl.ds(r0, ROWS_A)],
                            den_sp.at[pl.ds(r0, ROWS_A)])

        @pl.when(s == NSUB - 1)
        def _():
            r0 = (NSUB - 1) * ROWS_A
            pltpu.sync_copy(z128_hbm.at[pl.ds(r0, ROWS_LAST)],
                            acc_sp.at[pl.ds(r0, ROWS_LAST)])
            pltpu.sync_copy(z16_hbm.at[pl.ds(r0, ROWS_LAST)],
                            den_sp.at[pl.ds(r0, ROWS_LAST)])

        plsc.subcore_barrier()

        def idx_copies(t, b):
            base = (w + t * NTILE) * K
            return [
                (ei_hbm.at[0, pl.ds(base, K)], sbs[b], sem_i[b]),
                (ei_hbm.at[1, pl.ds(base, K)], dbs[b], sem_i[b]),
            ]

        def start_idx(t, b):
            tc = w + t * NTILE
            base = tc * K

            @pl.when(tc < SELF_CHUNK0)
            def _():
                for src, dst, sem in idx_copies(t, b):
                    pltpu.async_copy(src, dst, sem)

            @pl.when(tc >= SELF_CHUNK0)
            def _():
                @pl.loop(0, K // 16)
                def _(j):
                    v = (base - E + j * 16) + lax.iota(jnp.int32, 16)
                    sbs[b][pl.ds(j * 16, 16)] = jnp.minimum(v, N - 1)
                    dbs[b][pl.ds(j * 16, 16)] = jnp.minimum(v, TRASH)

        def wait_idx(t, b):
            tc = w + t * NTILE

            @pl.when(tc < SELF_CHUNK0)
            def _():
                for src, dst, sem in idx_copies(t, b):
                    pltpu.make_async_copy(src, dst, sem).wait()

        def gat_copies(b):
            return [
                (as_hbm.at[sbs[b]], ab, sem_g[b]),
                (ad_hbm.at[dbs[b]], bbs[b], sem_g[b]),
                (hw_hbm.at[sbs[b]], hbs[b], sem_g[b]),
            ]

        def start_gat(t, b):
            tc = w + t * NTILE
            for src, dst, sem in gat_copies(b):
                pltpu.async_copy(src, dst, sem)

            @pl.when(tc < SELF_CHUNK0)
            def _():
                pltpu.async_copy(p_hbm.at[pl.ds(tc * K * HEADS, K * HEADS)],
                                 aeb.at[pl.ds(0, K * HEADS)], sem_g[b])

            @pl.when(tc >= SELF_CHUNK0)
            def _():
                pltpu.async_copy(aep_hbm, aeb.at[pl.ds(0, K * HEADS)],
                                 sem_g[b])

        def wait_gat(b):
            for src, dst, sem in gat_copies(b):
                pltpu.make_async_copy(src, dst, sem).wait()
            pltpu.make_async_copy(aep_hbm, aeb.at[pl.ds(0, K * HEADS)],
                                  sem_g[b]).wait()

        def sc_copies(b):
            return [
                (hbs[b], acc_sp.at[dbs[b]], sem_s[b]),
                (bbs[b], den_sp.at[dbs[b]], sem_s[b]),
            ]

        _dn = lax.GatherDimensionNumbers(
            offset_dims=(), collapsed_slice_dims=(0,), start_index_map=(0,))

        def _splat(v, j):
            idx = jnp.full((16, 1), j, dtype=jnp.int32)
            return lax.gather(v, idx, _dn, (1,),
                              mode=lax.GatherScatterMode.PROMISE_IN_BOUNDS)

        def compute(b):
            @pl.loop(0, K)
            def _(i):
                al = ab[i, :] + bbs[b][i, :] + aeb[pl.ds(i * HEADS, 16)]
                al = jnp.where(al >= 0.0, al, al * 0.2)
                ex = jnp.exp(al)
                bbs[b][i, :] = ex
                for j in range(HEADS):
                    sl = pl.ds(j * 16, 16)
                    hbs[b][i, sl] = hbs[b][i, sl] * _splat(ex, j)

        start_idx(0, 0)
        wait_idx(0, 0)
        start_gat(0, 0)

        @pl.loop(0, CPT, step=2)
        def _(t0):
            for b in range(2):
                t = t0 + b
                b2 = 1 - b

                @pl.when(t >= 1)
                def _():
                    for src, dst, sem in sc_copies(b2):
                        pltpu.make_async_copy(src, dst, sem).wait()

                @pl.when(t + 1 < CPT)
                def _():
                    start_idx(t + 1, b2)

                wait_gat(b)

                compute(b)

                for src, dst, sem in sc_copies(b):
                    pltpu.async_copy(src, dst, sem, add=True)

                @pl.when(t + 1 < CPT)
                def _():
                    wait_idx(t + 1, b2)
                    start_gat(t + 1, b2)

        for src, dst, sem in sc_copies(1):
            pltpu.make_async_copy(src, dst, sem).wait()

        plsc.subcore_barrier()

        @pl.when(s < NSUB - 1)
        def _():
            r0 = s * ROWS_A
            pltpu.sync_copy(acc_sp.at[pl.ds(r0, ROWS_A)],
                            acc_out.at[c, pl.ds(r0, ROWS_A)])
            pltpu.sync_copy(den_sp.at[pl.ds(r0, ROWS_A)],
                            den_out.at[c, pl.ds(r0, ROWS_A)])

        @pl.when(s == NSUB - 1)
        def _():
            r0 = (NSUB - 1) * ROWS_A
            pltpu.sync_copy(acc_sp.at[pl.ds(r0, ROWS_LAST)],
                            acc_out.at[c, pl.ds(r0, ROWS_LAST)])
            pltpu.sync_copy(den_sp.at[pl.ds(r0, ROWS_LAST)],
                            den_out.at[c, pl.ds(r0, ROWS_LAST)])

    return k(hw, a_src16, a_dst16, edge_index, p_l, aepad_l, z128, z16)


def _att_fold(att):
    flat = att[0].reshape(HD)
    h_of = jnp.arange(HD, dtype=jnp.int32) // HID
    return flat[:, None] * jax.nn.one_hot(h_of, EDGE_DIM, dtype=jnp.float32)


def kernel(x, edge_index, edge_attr, params):
    layers = params["layers"]

    we_all = jnp.stack([
        jnp.sum(p["W_edge"].reshape(EDGE_DIM, HEADS, HID)
                * p["att_edge"][0][None], axis=-1)
        for p in layers])
    a_src_m = [_att_fold(p["att_src"]) for p in layers]
    a_dst_m = [_att_fold(p["att_dst"]) for p in layers]

    h_of = jnp.arange(HD, dtype=jnp.int32) // HID
    d_of = jnp.arange(HD, dtype=jnp.int32) % HID
    rmat = jax.nn.one_hot(h_of, EDGE_DIM, dtype=jnp.float32).T
    mmat = jax.nn.one_hot(d_of, HID, dtype=jnp.float32) / HEADS

    r2 = lambda v: v.reshape(1, -1)
    z128 = jnp.zeros((NP, HD), jnp.float32)
    z16 = jnp.zeros((NP, EDGE_DIM), jnp.float32)

    h, hw, s16, d16 = pl.pallas_call(
        _k_in_body,
        grid=(_GRID_N,),
        in_specs=[_rows((_ROWS_B, IN_DIM)), _full((IN_DIM, HD)),
                  _full((1, HD)), _full((1, HD)), _full((1, HD)),
                  _full((HD, HD)), _full((HD, EDGE_DIM)),
                  _full((HD, EDGE_DIM))],
        out_specs=[_rows((_ROWS_B, HD)), _rows((_ROWS_B, HD)),
                   _rows((_ROWS_B, EDGE_DIM)), _rows((_ROWS_B, EDGE_DIM))],
        out_shape=[jax.ShapeDtypeStruct((N, HD), jnp.float32),
                   jax.ShapeDtypeStruct((N, HD), jnp.float32),
                   jax.ShapeDtypeStruct((N, EDGE_DIM), jnp.float32),
                   jax.ShapeDtypeStruct((N, EDGE_DIM), jnp.float32)],
    )(x, params["W_in"], r2(params["b_in"]),
      r2(params["ln_in_g"]), r2(params["ln_in_b"]),
      layers[0]["W"], a_src_m[0], a_dst_m[0])

    p_all = [
        pl.pallas_call(
            _k_edge_body,
            grid=(_GRID_E,),
            in_specs=[_rows((_EB, EDGE_DIM)), _full((EDGE_DIM, HEADS))],
            out_specs=_rows((_EB, HEADS)),
            out_shape=jax.ShapeDtypeStruct((E, HEADS), jnp.float32),
        )(edge_attr, we_all[l])
        for l in range(NUM_LAYERS)
    ]

    msum = pl.pallas_call(
        _k_mean_body,
        grid=(_GRID_E,),
        in_specs=[_rows((_EB, EDGE_DIM))],
        out_specs=_full((8, EDGE_DIM)),
        out_shape=jax.ShapeDtypeStruct((8, EDGE_DIM), jnp.float32),
    )(edge_attr)

    aepad = pl.pallas_call(
        _k_pad_body,
        in_specs=[pl.BlockSpec((8, EDGE_DIM), lambda: (0, 0)),
                  pl.BlockSpec((NUM_LAYERS, EDGE_DIM, HEADS),
                               lambda: (0, 0, 0))],
        out_specs=pl.BlockSpec((NUM_LAYERS, K, HEADS), lambda: (0, 0, 0)),
        out_shape=jax.ShapeDtypeStruct((NUM_LAYERS, K, HEADS), jnp.float32),
    )(msum, we_all)
    aepad = aepad.reshape(NUM_LAYERS, K * HEADS)

    for l in range(NUM_LAYERS):
        p = layers[l]
        d16p = jnp.concatenate(
            [d16, jnp.zeros((NP - N, EDGE_DIM), jnp.float32)], axis=0)

        acc, den = _sc_layer(hw, s16, d16p, edge_index,
                             p_all[l].reshape(-1), aepad[l], z128, z16)

        if l < NUM_LAYERS - 1:
            pn = layers[l + 1]
            h, hw, s16, d16 = pl.pallas_call(
                functools.partial(_k_fused_body, use_res=(l > 0)),
                grid=(_GRID_N,),
                in_specs=[
                    pl.BlockSpec((NCORE, _ROWS_B, HD), lambda i: (0, i, 0)),
                    pl.BlockSpec((NCORE, _ROWS_B, EDGE_DIM),
                                 lambda i: (0, i, 0)),
                    _full((EDGE_DIM, HD)), _full((1, HD)), _full((1, HD)),
                    _full((1, HD)), _rows((_ROWS_B, HD)),
                    _full((HD, HD)), _full((HD, EDGE_DIM)),
                    _full((HD, EDGE_DIM))],
                out_specs=[_rows((_ROWS_B, HD)), _rows((_ROWS_B, HD)),
                           _rows((_ROWS_B, EDGE_DIM)),
                           _rows((_ROWS_B, EDGE_DIM))],
                out_shape=[jax.ShapeDtypeStruct((N, HD), jnp.float32),
                           jax.ShapeDtypeStruct((N, HD), jnp.float32),
                           jax.ShapeDtypeStruct((N, EDGE_DIM), jnp.float32),
                           jax.ShapeDtypeStruct((N, EDGE_DIM), jnp.float32)],
            )(acc, den, rmat, r2(p["bias"]), r2(p["ln_g"]), r2(p["ln_b"]),
              h, pn["W"], a_src_m[l + 1], a_dst_m[l + 1])
        else:
            h = pl.pallas_call(
                _k_final_body,
                grid=(_GRID_N,),
                in_specs=[
                    pl.BlockSpec((NCORE, _ROWS_B, HD), lambda i: (0, i, 0)),
                    pl.BlockSpec((NCORE, _ROWS_B, EDGE_DIM),
                                 lambda i: (0, i, 0)),
                    _full((EDGE_DIM, HD)), _full((HD, HID)),
                    _full((1, HID)), _full((1, HID)), _full((1, HID))],
                out_specs=_rows((_ROWS_B, HID)),
                out_shape=jax.ShapeDtypeStruct((N, HID), jnp.float32),
            )(acc, den, rmat, mmat, r2(p["bias"]), r2(p["ln_g"]),
              r2(p["ln_b"]))

    return h


# speedup vs baseline: 1.6911x; 1.0635x over previous
import dataclasses
import functools

import jax
import jax.numpy as jnp
from jax import lax
from jax.experimental import pallas as pl
from jax.experimental.pallas import tpu as pltpu
from jax.experimental.pallas import tpu_sc as plsc

N = 10000
E = 320000
IN_DIM = 128
HID = 16
HEADS = 8
NUM_LAYERS = 4
EDGE_DIM = 16
HD = HID * HEADS

NP = 10008
TRASH = N
K = 128
NCORE = 2
NSUB = 16
NTILE = NCORE * NSUB
ETOT = E + N
CHUNKS = 2624
CPT = CHUNKS // NTILE
SELF_CHUNK0 = E // K
ROWS_A = 632
ROWS_LAST = NP - 15 * ROWS_A

_HIGH = lax.Precision.HIGHEST


def _dot(a, b):
    return jnp.dot(a, b, preferred_element_type=jnp.float32)


def _layer_norm(h, g, bb):
    m = jnp.mean(h, axis=-1, keepdims=True)
    v = jnp.mean((h - m) ** 2, axis=-1, keepdims=True)
    return (h - m) / jnp.sqrt(v + 1e-5) * g + bb


def _k_in_body(x_ref, w_ref, b_ref, g_ref, bb_ref, wn_ref, as_ref, ad_ref,
               h_ref, hw_ref, s16_ref, d16_ref):
    h = jnp.maximum(
        _layer_norm(_dot(x_ref[...], w_ref[...]) + b_ref[...],
                    g_ref[...], bb_ref[...]), 0.0)
    h_ref[...] = h
    hw = _dot(h, wn_ref[...])
    hw_ref[...] = hw
    s16_ref[...] = _dot(hw, as_ref[...])
    d16_ref[...] = _dot(hw, ad_ref[...])


def _k_edge_body(ea_ref, we_ref, p_ref):
    p_ref[...] = _dot(ea_ref[...], we_ref[...])


def _k_mean_body(ea_ref, o_ref):
    @pl.when(pl.program_id(0) == 0)
    def _():
        o_ref[...] = jnp.zeros_like(o_ref)

    s = jnp.sum(ea_ref[...], axis=0, keepdims=True)
    o_ref[...] += jnp.broadcast_to(s, o_ref.shape)


def _k_pad_body(ms_ref, we_ref, o_ref):
    m = ms_ref[0:1, :] * (1.0 / E)
    for l in range(NUM_LAYERS):
        v = _dot(m, we_ref[l])
        o_ref[l] = jnp.broadcast_to(v, (K, HEADS))


def _k_fused_body(acc_ref, den_ref, r_ref, b_ref, g_ref, bb_ref, res_ref,
                  wn_ref, as_ref, ad_ref,
                  h_ref, hw_ref, s16_ref, d16_ref, *, use_res):
    a = acc_ref[0] + acc_ref[1]
    den = den_ref[0] + den_ref[1] + 1e-16
    dene = _dot(den, r_ref[...])
    o = _layer_norm(a / dene + b_ref[...], g_ref[...], bb_ref[...])
    if use_res:
        o = o + res_ref[...]
    h = jnp.where(o > 0, o, jnp.exp(o) - 1.0)
    h_ref[...] = h
    hw = _dot(h, wn_ref[...])
    hw_ref[...] = hw
    s16_ref[...] = _dot(hw, as_ref[...])
    d16_ref[...] = _dot(hw, ad_ref[...])


def _k_final_body(acc_ref, den_ref, r_ref, m_ref, b_ref, g_ref, bb_ref,
                  o_ref):
    a = acc_ref[0] + acc_ref[1]
    den = den_ref[0] + den_ref[1] + 1e-16
    dene = _dot(den, r_ref[...])
    o = _dot(a / dene, m_ref[...]) + b_ref[...]
    o_ref[...] = _layer_norm(o, g_ref[...], bb_ref[...])


_ROWS_B = 1000
_GRID_N = N // _ROWS_B
_EB = 4000
_GRID_E = E // _EB


def _full(shape):
    return pl.BlockSpec(shape, lambda i: (0,) * len(shape))


def _rows(shape):
    return pl.BlockSpec(shape, lambda i: (i,) + (0,) * (len(shape) - 1))


def _sc_layer(hw, a_src16, a_dst16, edge_index, p_l, aepad_l, z128, z16):
    mesh = plsc.VectorSubcoreMesh(
        core_axis_name="c", subcore_axis_name="s",
        num_cores=NCORE, num_subcores=NSUB)

    cp = pltpu.CompilerParams()
    if "needs_layout_passes" in pltpu.CompilerParams.__dataclass_fields__:
        cp = dataclasses.replace(cp, needs_layout_passes=False)
    if "use_tc_tiling_on_sc" in pltpu.CompilerParams.__dataclass_fields__:
        cp = dataclasses.replace(cp, use_tc_tiling_on_sc=False)

    @functools.partial(
        pl.kernel,
        out_type=[
            jax.ShapeDtypeStruct((NCORE, NP, HD), jnp.float32),
            jax.ShapeDtypeStruct((NCORE, NP, EDGE_DIM), jnp.float32),
        ],
        mesh=mesh,
        scratch_types=[
            pltpu.VMEM_SHARED((NP, HD), jnp.float32),
            pltpu.VMEM_SHARED((NP, EDGE_DIM), jnp.float32),
            [pltpu.VMEM((K, HD), jnp.float32)] * 2,
            pltpu.VMEM((K, EDGE_DIM), jnp.float32),
            [pltpu.VMEM((K, EDGE_DIM), jnp.float32)] * 2,
            pltpu.VMEM((K * HEADS + 16,), jnp.float32),
            [pltpu.VMEM((K,), jnp.int32)] * 2,
            [pltpu.VMEM((K,), jnp.int32)] * 2,
            [pltpu.SemaphoreType.DMA] * 2,
            [pltpu.SemaphoreType.DMA] * 2,
            [pltpu.SemaphoreType.DMA] * 2,
        ],
        compiler_params=cp,
    )
    def k(hw_hbm, as_hbm, ad_hbm, ei_hbm, p_hbm, aep_hbm, z128_hbm, z16_hbm,
          acc_out, den_out,
          acc_sp, den_sp, hbs, ab, bbs, aeb, sbs, dbs,
          sem_i, sem_g, sem_s):
        c = lax.axis_index("c")
        s = lax.axis_index("s")
        w = c * NSUB + s

        @pl.when(s < NSUB - 1)
        def _():
            r0 = s * ROWS_A
            pltpu.sync_copy(z128_hbm.at[pl.ds(r0, ROWS_A)],
                            acc_sp.at[pl.ds(r0, ROWS_A)])
            pltpu.sync_copy(z16_hbm.at[pl.ds(r0, ROWS_A)],
                            den_sp.at[pl.ds(r0, ROWS_A)])

        @pl.when(s == NSUB - 1)
        def _():
            r0 = (NSUB - 1) * ROWS_A
            pltpu.sync_copy(z128_hbm.at[pl.ds(r0, ROWS_LAST)],
                            acc_sp.at[pl.ds(r0, ROWS_LAST)])
            pltpu.sync_copy(z16_hbm.at[pl.ds(r0, ROWS_LAST)],
                            den_sp.at[pl.ds(r0, ROWS_LAST)])

        plsc.subcore_barrier()

        def idx_copies(t, b):
            base = (w + t * NTILE) * K
            return [
                (ei_hbm.at[0, pl.ds(base, K)], sbs[b], sem_i[b]),
                (ei_hbm.at[1, pl.ds(base, K)], dbs[b], sem_i[b]),
            ]

        def start_idx(t, b):
            tc = w + t * NTILE
            base = tc * K

            @pl.when(tc < SELF_CHUNK0)
            def _():
                for src, dst, sem in idx_copies(t, b):
                    pltpu.async_copy(src, dst, sem)

            @pl.when(tc >= SELF_CHUNK0)
            def _():
                @pl.loop(0, K // 16)
                def _(j):
                    v = (base - E + j * 16) + lax.iota(jnp.int32, 16)
                    sbs[b][pl.ds(j * 16, 16)] = jnp.minimum(v, N - 1)
                    dbs[b][pl.ds(j * 16, 16)] = jnp.minimum(v, TRASH)

        def wait_idx(t, b):
            tc = w + t * NTILE

            @pl.when(tc < SELF_CHUNK0)
            def _():
                for src, dst, sem in idx_copies(t, b):
                    pltpu.make_async_copy(src, dst, sem).wait()

        def gat_copies(b):
            return [
                (as_hbm.at[sbs[b]], ab, sem_g[b]),
                (ad_hbm.at[dbs[b]], bbs[b], sem_g[b]),
                (hw_hbm.at[sbs[b]], hbs[b], sem_g[b]),
            ]

        def start_gat(t, b):
            tc = w + t * NTILE
            for src, dst, sem in gat_copies(b):
                pltpu.async_copy(src, dst, sem)

            @pl.when(tc < SELF_CHUNK0)
            def _():
                pltpu.async_copy(p_hbm.at[pl.ds(tc * K * HEADS, K * HEADS)],
                                 aeb.at[pl.ds(0, K * HEADS)], sem_g[b])

            @pl.when(tc >= SELF_CHUNK0)
            def _():
                pltpu.async_copy(aep_hbm, aeb.at[pl.ds(0, K * HEADS)],
                                 sem_g[b])

        def wait_gat(b):
            for src, dst, sem in gat_copies(b):
                pltpu.make_async_copy(src, dst, sem).wait()
            pltpu.make_async_copy(aep_hbm, aeb.at[pl.ds(0, K * HEADS)],
                                  sem_g[b]).wait()

        def sc_copies(b):
            return [
                (hbs[b], acc_sp.at[dbs[b]], sem_s[b]),
                (bbs[b], den_sp.at[dbs[b]], sem_s[b]),
            ]

        _dn = lax.GatherDimensionNumbers(
            offset_dims=(), collapsed_slice_dims=(0,), start_index_map=(0,))

        def _splat(v, j):
            idx = jnp.full((16, 1), j, dtype=jnp.int32)
            return lax.gather(v, idx, _dn, (1,),
                              mode=lax.GatherScatterMode.PROMISE_IN_BOUNDS)

        def compute(b):
            @pl.loop(0, K)
            def _(i):
                al = ab[i, :] + bbs[b][i, :] + aeb[pl.ds(i * HEADS, 16)]
                al = jnp.where(al >= 0.0, al, al * 0.2)
                ex = jnp.exp(al)
                bbs[b][i, :] = ex
                for j in range(HEADS):
                    sl = pl.ds(j * 16, 16)
                    hbs[b][i, sl] = hbs[b][i, sl] * _splat(ex, j)

        start_idx(0, 0)
        wait_idx(0, 0)
        start_gat(0, 0)

        @pl.loop(0, CPT, step=2)
        def _(t0):
            for b in range(2):
                t = t0 + b
                b2 = 1 - b

                @pl.when(t >= 1)
                def _():
                    for src, dst, sem in sc_copies(b2):
                        pltpu.make_async_copy(src, dst, sem).wait()

                @pl.when(t + 1 < CPT)
                def _():
                    start_idx(t + 1, b2)

                wait_gat(b)

                compute(b)

                for src, dst, sem in sc_copies(b):
                    pltpu.async_copy(src, dst, sem, add=True)

                @pl.when(t + 1 < CPT)
                def _():
                    wait_idx(t + 1, b2)
                    start_gat(t + 1, b2)

        for src, dst, sem in sc_copies(1):
            pltpu.make_async_copy(src, dst, sem).wait()

        plsc.subcore_barrier()

        @pl.when(s < NSUB - 1)
        def _():
            r0 = s * ROWS_A
            pltpu.sync_copy(acc_sp.at[pl.ds(r0, ROWS_A)],
                            acc_out.at[c, pl.ds(r0, ROWS_A)])
            pltpu.sync_copy(den_sp.at[pl.ds(r0, ROWS_A)],
                            den_out.at[c, pl.ds(r0, ROWS_A)])

        @pl.when(s == NSUB - 1)
        def _():
            r0 = (NSUB - 1) * ROWS_A
            pltpu.sync_copy(acc_sp.at[pl.ds(r0, ROWS_LAST)],
                            acc_out.at[c, pl.ds(r0, ROWS_LAST)])
            pltpu.sync_copy(den_sp.at[pl.ds(r0, ROWS_LAST)],
                            den_out.at[c, pl.ds(r0, ROWS_LAST)])

    return k(hw, a_src16, a_dst16, edge_index, p_l, aepad_l, z128, z16)


def _att_fold(att):
    flat = att[0].reshape(HD)
    h_of = jnp.arange(HD, dtype=jnp.int32) // HID
    return flat[:, None] * jax.nn.one_hot(h_of, EDGE_DIM, dtype=jnp.float32)


def kernel(x, edge_index, edge_attr, params):
    layers = params["layers"]

    we_all = jnp.stack([
        jnp.sum(p["W_edge"].reshape(EDGE_DIM, HEADS, HID)
                * p["att_edge"][0][None], axis=-1)
        for p in layers])
    a_src_m = [_att_fold(p["att_src"]) for p in layers]
    a_dst_m = [_att_fold(p["att_dst"]) for p in layers]

    h_of = jnp.arange(HD, dtype=jnp.int32) // HID
    d_of = jnp.arange(HD, dtype=jnp.int32) % HID
    rmat = jax.nn.one_hot(h_of, EDGE_DIM, dtype=jnp.float32).T
    mmat = jax.nn.one_hot(d_of, HID, dtype=jnp.float32) / HEADS

    r2 = lambda v: v.reshape(1, -1)
    z128 = jnp.zeros((NP, HD), jnp.float32)
    z16 = jnp.zeros((NP, EDGE_DIM), jnp.float32)

    h, hw, s16, d16 = pl.pallas_call(
        _k_in_body,
        grid=(_GRID_N,),
        in_specs=[_rows((_ROWS_B, IN_DIM)), _full((IN_DIM, HD)),
                  _full((1, HD)), _full((1, HD)), _full((1, HD)),
                  _full((HD, HD)), _full((HD, EDGE_DIM)),
                  _full((HD, EDGE_DIM))],
        out_specs=[_rows((_ROWS_B, HD)), _rows((_ROWS_B, HD)),
                   _rows((_ROWS_B, EDGE_DIM)), _rows((_ROWS_B, EDGE_DIM))],
        out_shape=[jax.ShapeDtypeStruct((N, HD), jnp.float32),
                   jax.ShapeDtypeStruct((N, HD), jnp.float32),
                   jax.ShapeDtypeStruct((N, EDGE_DIM), jnp.float32),
                   jax.ShapeDtypeStruct((N, EDGE_DIM), jnp.float32)],
    )(x, params["W_in"], r2(params["b_in"]),
      r2(params["ln_in_g"]), r2(params["ln_in_b"]),
      layers[0]["W"], a_src_m[0], a_dst_m[0])

    p_all = [
        pl.pallas_call(
            _k_edge_body,
            grid=(_GRID_E,),
            in_specs=[_rows((_EB, EDGE_DIM)), _full((EDGE_DIM, HEADS))],
            out_specs=_rows((_EB, HEADS)),
            out_shape=jax.ShapeDtypeStruct((E, HEADS), jnp.float32),
        )(edge_attr, we_all[l])
        for l in range(NUM_LAYERS)
    ]

    msum = pl.pallas_call(
        _k_mean_body,
        grid=(_GRID_E,),
        in_specs=[_rows((_EB, EDGE_DIM))],
        out_specs=_full((8, EDGE_DIM)),
        out_shape=jax.ShapeDtypeStruct((8, EDGE_DIM), jnp.float32),
    )(edge_attr)

    aepad = pl.pallas_call(
        _k_pad_body,
        in_specs=[pl.BlockSpec((8, EDGE_DIM), lambda: (0, 0)),
                  pl.BlockSpec((NUM_LAYERS, EDGE_DIM, HEADS),
                               lambda: (0, 0, 0))],
        out_specs=pl.BlockSpec((NUM_LAYERS, K, HEADS), lambda: (0, 0, 0)),
        out_shape=jax.ShapeDtypeStruct((NUM_LAYERS, K, HEADS), jnp.float32),
    )(msum, we_all)
    aepad = aepad.reshape(NUM_LAYERS, K * HEADS)

    for l in range(NUM_LAYERS):
        p = layers[l]
        d16p = jnp.concatenate(
            [d16, jnp.zeros((NP - N, EDGE_DIM), jnp.float32)], axis=0)

        acc, den = _sc_layer(hw, s16, d16p, edge_index,
                             p_all[l].reshape(-1), aepad[l], z128, z16)

        if l < NUM_LAYERS - 1:
            pn = layers[l + 1]
            h, hw, s16, d16 = pl.pallas_call(
                functools.partial(_k_fused_body, use_res=(l > 0)),
                grid=(_GRID_N,),
                in_specs=[
                    pl.BlockSpec((NCORE, _ROWS_B, HD), lambda i: (0, i, 0)),
                    pl.BlockSpec((NCORE, _ROWS_B, EDGE_DIM),
                                 lambda i: (0, i, 0)),
                    _full((EDGE_DIM, HD)), _full((1, HD)), _full((1, HD)),
                    _full((1, HD)), _rows((_ROWS_B, HD)),
                    _full((HD, HD)), _full((HD, EDGE_DIM)),
                    _full((HD, EDGE_DIM))],
                out_specs=[_rows((_ROWS_B, HD)), _rows((_ROWS_B, HD)),
                           _rows((_ROWS_B, EDGE_DIM)),
                           _rows((_ROWS_B, EDGE_DIM))],
                out_shape=[jax.ShapeDtypeStruct((N, HD), jnp.float32),
                           jax.ShapeDtypeStruct((N, HD), jnp.float32),
                           jax.ShapeDtypeStruct((N, EDGE_DIM), jnp.float32),
                           jax.ShapeDtypeStruct((N, EDGE_DIM), jnp.float32)],
            )(acc, den, rmat, r2(p["bias"]), r2(p["ln_g"]), r2(p["ln_b"]),
              h, pn["W"], a_src_m[l + 1], a_dst_m[l + 1])
        else:
            h = pl.pallas_call(
                _k_final_body,
                grid=(_GRID_N,),
                in_specs=[
                    pl.BlockSpec((NCORE, _ROWS_B, HD), lambda i: (0, i, 0)),
                    pl.BlockSpec((NCORE, _ROWS_B, EDGE_DIM),
                                 lambda i: (0, i, 0)),
                    _full((EDGE_DIM, HD)), _full((HD, HID)),
                    _full((1, HID)), _full((1, HID)), _full((1, HID))],
                out_specs=_rows((_ROWS_B, HID)),
                out_shape=jax.ShapeDtypeStruct((N, HID), jnp.float32),
            )(acc, den, rmat, mmat, r2(p["bias"]), r2(p["ln_g"]),
              r2(p["ln_b"]))

    return h
